# baseline (speedup 1.0000x reference)
import jax
import jax.numpy as jnp
from jax.experimental import pallas as pl
from jax.experimental.pallas import tpu as pltpu

SIZE = 1024
SMOOTHING = 0.1
CONFIDENCE = 1.0 - SMOOTHING
T = 16384

ROWS = 512
NBLK = T // ROWS
LOG19 = 0.6418538861723947


def _reduce_kernel(x_ref, tgt_ref, mask_ref, out_ref):
    x = x_ref[...]
    m = mask_ref[...]
    t = tgt_ref[0, 0, :]

    mf = m.astype(jnp.float32)
    col = jax.lax.broadcasted_iota(jnp.int32, (ROWS, SIZE), 1)
    onehot = col == t[:, None]

    rowsum = jnp.sum(x, axis=1)
    sv = jnp.sum(jnp.where(m, x, 0.0), axis=1)
    mv = jnp.sum(mf, axis=1)
    x_t = jnp.sum(jnp.where(onehot, x, 0.0), axis=1)
    v_t = jnp.sum(jnp.where(onehot, mf, 0.0), axis=1)

    packed = jnp.concatenate(
        [q.reshape(1, 1, 1, ROWS) for q in (rowsum, sv, mv, x_t, v_t)], axis=0
    )
    out_ref[...] = packed


def _combine_kernel(red_ref, out_ref):
    q = red_ref[...]
    rowsum = q[0, :, 0, :]
    sv = q[1, :, 0, :]
    mv = q[2, :, 0, :]
    x_t = q[3, :, 0, :]
    v_t = q[4, :, 0, :]

    su = rowsum - sv
    cnt = jnp.float32(SIZE) - mv
    has_unv = cnt > 0.0
    s = SMOOTHING / jnp.maximum(cnt, 1.0)
    log_s = jnp.log(s)
    base = -sv + jnp.where(has_unv, SMOOTHING * log_s - s * su, 0.0)

    corr_vis = jnp.float32(1.9 * LOG19) - 0.9 * x_t
    sp = s + CONFIDENCE
    corr_unv = sp * jnp.log(sp) - s * log_s - 0.9 * x_t
    corr = jnp.where(v_t > 0.5, corr_vis, corr_unv)

    out_ref[...] = jnp.sum(base + corr).reshape(1, 1)


@jax.jit
def kernel(x, target, visited_mask):
    tgt3 = target.reshape(NBLK, 1, ROWS)
    red = pl.pallas_call(
        _reduce_kernel,
        grid=(NBLK,),
        in_specs=[
            pl.BlockSpec((ROWS, SIZE), lambda i: (i, 0)),
            pl.BlockSpec((1, 1, ROWS), lambda i: (i, 0, 0)),
            pl.BlockSpec((ROWS, SIZE), lambda i: (i, 0)),
        ],
        out_specs=pl.BlockSpec((5, 1, 1, ROWS), lambda i: (0, i, 0, 0)),
        out_shape=jax.ShapeDtypeStruct((5, NBLK, 1, ROWS), jnp.float32),
        compiler_params=pltpu.CompilerParams(
            dimension_semantics=("parallel",),
        ),
    )(x, tgt3, visited_mask)

    out = pl.pallas_call(
        _combine_kernel,
        out_shape=jax.ShapeDtypeStruct((1, 1), jnp.float32),
    )(red)
    return out[0, 0]
